# baseline (speedup 1.0000x reference)
import jax
import jax.numpy as jnp
from jax import lax
from jax.experimental import pallas as pl
from jax.experimental.pallas import tpu as pltpu
from jax.experimental.pallas import tpu_sc as plsc

_B = 16384
_D = 32
_NC = 2
_NS = 16
_NW = _NC * _NS
_BPW = _B // _NW
_CH = 128
_NCHUNK = _BPW // _CH
_L = 16


def _cf_body(uidx_hbm, midx_hbm, uf_hbm, ub_hbm, mf_hbm, mb_hbm, out_hbm,
             uidx_v, midx_v, uf_v, mf_v, ub_v, mb_v, out_v, sem):
    wid = lax.axis_index("s") * _NC + lax.axis_index("c")
    base = wid * _BPW

    for j in range(_NCHUNK):
        pltpu.sync_copy(uidx_hbm.at[pl.ds(base + j * _CH, _CH)], uidx_v.at[j])
        pltpu.sync_copy(midx_hbm.at[pl.ds(base + j * _CH, _CH)], midx_v.at[j])

    cps = []
    for j in range(_NCHUNK):
        sl = pl.ds(j * _CH, _CH)
        cps.append(pltpu.async_copy(uf_hbm.at[uidx_v.at[j]], uf_v.at[sl], sem))
        cps.append(pltpu.async_copy(mf_hbm.at[midx_v.at[j]], mf_v.at[sl], sem))
        cps.append(pltpu.async_copy(ub_hbm.at[uidx_v.at[j]], ub_v.at[sl], sem))
        cps.append(pltpu.async_copy(mb_hbm.at[midx_v.at[j]], mb_v.at[sl], sem))
    for cp in cps:
        cp.wait()

    lanes = jnp.arange(_L, dtype=jnp.int32)
    for blk in range(_BPW // _L):
        rows = lanes + (blk * _L)
        accs = [
            ub_v[pl.ds(blk * _L, _L)] + mb_v[pl.ds(blk * _L, _L)],
            jnp.zeros((_L,), jnp.float32),
            jnp.zeros((_L,), jnp.float32),
            jnp.zeros((_L,), jnp.float32),
        ]
        for d in range(_D):
            dv = jnp.full((_L,), d, jnp.int32)
            u = plsc.load_gather(uf_v, [rows, dv])
            m = plsc.load_gather(mf_v, [rows, dv])
            accs[d % 4] = accs[d % 4] + u * m
        out_v[pl.ds(blk * _L, _L)] = (
            (accs[0] + accs[1]) + (accs[2] + accs[3]))

    pltpu.sync_copy(out_v, out_hbm.at[pl.ds(base, _BPW)])


@jax.jit
def _cf_call(u_idx, m_idx, user_factors, user_bias, movie_factors, movie_bias):
    mesh = plsc.VectorSubcoreMesh(core_axis_name="c", subcore_axis_name="s",
                                  num_cores=_NC, num_subcores=_NS)
    return pl.kernel(
        _cf_body,
        out_type=jax.ShapeDtypeStruct((_B,), jnp.float32),
        mesh=mesh,
        scratch_types=[
            pltpu.VMEM((_NCHUNK, _CH), jnp.int32),
            pltpu.VMEM((_NCHUNK, _CH), jnp.int32),
            pltpu.VMEM((_BPW, _D), jnp.float32),
            pltpu.VMEM((_BPW, _D), jnp.float32),
            pltpu.VMEM((_BPW,), jnp.float32),
            pltpu.VMEM((_BPW,), jnp.float32),
            pltpu.VMEM((_BPW,), jnp.float32),
            pltpu.SemaphoreType.DMA,
        ],
        compiler_params=pltpu.CompilerParams(
            needs_layout_passes=False, use_tc_tiling_on_sc=False),
    )(u_idx, m_idx, user_factors, user_bias, movie_factors, movie_bias)


def kernel(x, user_factors, user_bias, movie_factors, movie_bias):
    u_idx = x[:, 0].astype(jnp.int32)
    m_idx = x[:, 1].astype(jnp.int32)
    return _cf_call(u_idx, m_idx, user_factors, user_bias[:, 0],
                    movie_factors, movie_bias[:, 0])
